# baseline (device time: 28865 ns/iter reference)
import jax
import jax.numpy as jnp
from jax import lax
from jax.experimental import pallas as pl
from jax.experimental.pallas import tpu as pltpu

N_DEV = 4
B_LOC = 2
SQ = 256
SKV = 256
HQ = 16
H_LOC = HQ // N_DEV
DH = 64
D_MODEL = 512
D_QK = HQ * DH
H_CHUNK = D_QK // N_DEV
BF = jnp.bfloat16


def kernel(x, Wq, K_ext, V_ext, Wo):
    pos = lax.axis_index("i")
    Kt = K_ext.transpose(0, 2, 3, 1)
    Vt = V_ext.transpose(0, 2, 3, 1)

    def body(x_ref, wq_ref, k_ref, v_ref, wo_ref, out_ref,
             wq_g, wo_g, x_scr, ctx_scr, k_scr, v_scr, kb_scr, vb_scr,
             kv_sem, wq_ss, wq_rs, wo_ss, wo_rs):
        my = lax.axis_index("i")
        left = lax.rem(my + (N_DEV - 1), N_DEV)
        right = lax.rem(my + 1, N_DEV)
        opp = lax.rem(my + 2, N_DEV)

        kv_dmas = []
        for b in range(B_LOC):
            bg = my * B_LOC + b
            kv_dmas.append(pltpu.make_async_copy(
                k_ref.at[bg], k_scr.at[b], kv_sem))
            kv_dmas.append(pltpu.make_async_copy(
                v_ref.at[bg], v_scr.at[b], kv_sem))
        for dma in kv_dmas:
            dma.start()

        barrier = pltpu.get_barrier_semaphore()
        for nbr in (left, right):
            pl.semaphore_signal(
                barrier, inc=1,
                device_id=(nbr,), device_id_type=pl.DeviceIdType.MESH,
            )
        pl.semaphore_wait(barrier, 2)

        row = lax.broadcasted_iota(jnp.int32, (SQ, SKV), 0) // 64
        col = lax.broadcasted_iota(jnp.int32, (SQ, SKV), 1) // 64
        mask = (row == col) | (col == 0) | (lax.rem(row + col, 3) == 0)
        neg = jnp.float32(-1e9)

        def mm(a, b):
            return lax.dot_general(
                a, b, dimension_numbers=(((1,), (0,)), ((), ())),
                preferred_element_type=jnp.float32,
            )

        def mm_t(a, b):
            return lax.dot_general(
                a, b, dimension_numbers=(((1,), (1,)), ((), ())),
                preferred_element_type=jnp.float32,
            )

        def do_chunk(r, wq_c, wo_c, first):
            for b in range(B_LOC):
                qp = mm(x_scr[b], wq_c)
                for hh in range(H_LOC):
                    h = lax.rem(my * H_LOC + r * H_LOC + hh, HQ)
                    q = (qp[:, hh * DH:(hh + 1) * DH]
                         * jnp.float32(0.125)).astype(BF)
                    kt = kb_scr[b, h]
                    s = mm(q, kt)
                    w = jnp.exp(jnp.where(mask, s, neg))
                    inv = 1.0 / jnp.sum(w, axis=1, keepdims=True)
                    w = (w * inv).astype(BF)
                    ctx_scr[:, hh * DH:(hh + 1) * DH] = mm_t(
                        w, vb_scr[b, h]).astype(BF)
                partial = mm(ctx_scr[...], wo_c)
                if first:
                    out_ref[b, :, :] = partial
                else:
                    out_ref[b, :, :] = out_ref[b, :, :] + partial

        wq_g[0] = wq_ref[...].astype(BF)
        wo_g[0] = wo_ref[...].astype(BF)

        hop0 = []
        for d, tgt, slot in ((0, right, 3), (1, left, 1)):
            rq = pltpu.make_async_remote_copy(
                src_ref=wq_g.at[0], dst_ref=wq_g.at[slot],
                send_sem=wq_ss.at[0, d], recv_sem=wq_rs.at[0, d],
                device_id=(tgt,), device_id_type=pl.DeviceIdType.MESH,
            )
            ro = pltpu.make_async_remote_copy(
                src_ref=wo_g.at[0], dst_ref=wo_g.at[slot],
                send_sem=wo_ss.at[0, d], recv_sem=wo_rs.at[0, d],
                device_id=(tgt,), device_id_type=pl.DeviceIdType.MESH,
            )
            rq.start()
            ro.start()
            hop0 += [rq, ro]

        for b in range(B_LOC):
            x_scr[b] = x_ref[b].astype(BF)

        for dma in kv_dmas:
            dma.wait()
        kb_scr[...] = k_scr[...].astype(BF)
        vb_scr[...] = v_scr[...].astype(BF)

        do_chunk(0, wq_g[0], wo_g[0], first=True)

        for r in hop0:
            r.wait()

        hop1 = []
        rq = pltpu.make_async_remote_copy(
            src_ref=wq_g.at[3, pl.ds(0, D_MODEL // 2)],
            dst_ref=wq_g.at[2, pl.ds(0, D_MODEL // 2)],
            send_sem=wq_ss.at[1, 0], recv_sem=wq_rs.at[1, 0],
            device_id=(right,), device_id_type=pl.DeviceIdType.MESH,
        )
        ro = pltpu.make_async_remote_copy(
            src_ref=wo_g.at[3, pl.ds(0, H_CHUNK // 2)],
            dst_ref=wo_g.at[2, pl.ds(0, H_CHUNK // 2)],
            send_sem=wo_ss.at[1, 0], recv_sem=wo_rs.at[1, 0],
            device_id=(right,), device_id_type=pl.DeviceIdType.MESH,
        )
        rq.start()
        ro.start()
        hop1 += [rq, ro]
        rq = pltpu.make_async_remote_copy(
            src_ref=wq_g.at[1, pl.ds(D_MODEL // 2, D_MODEL // 2)],
            dst_ref=wq_g.at[2, pl.ds(D_MODEL // 2, D_MODEL // 2)],
            send_sem=wq_ss.at[1, 1], recv_sem=wq_rs.at[1, 1],
            device_id=(left,), device_id_type=pl.DeviceIdType.MESH,
        )
        ro = pltpu.make_async_remote_copy(
            src_ref=wo_g.at[1, pl.ds(H_CHUNK // 2, H_CHUNK // 2)],
            dst_ref=wo_g.at[2, pl.ds(H_CHUNK // 2, H_CHUNK // 2)],
            send_sem=wo_ss.at[1, 1], recv_sem=wo_rs.at[1, 1],
            device_id=(left,), device_id_type=pl.DeviceIdType.MESH,
        )
        rq.start()
        ro.start()
        hop1 += [rq, ro]

        do_chunk(3, wq_g[3], wo_g[3], first=False)
        do_chunk(1, wq_g[1], wo_g[1], first=False)

        for r in hop1:
            r.wait()

        do_chunk(2, wq_g[2], wo_g[2], first=False)

    return pl.pallas_call(
        body,
        out_shape=jax.ShapeDtypeStruct((B_LOC, SQ, D_MODEL), jnp.float32),
        in_specs=[
            pl.BlockSpec(memory_space=pltpu.VMEM),
            pl.BlockSpec(memory_space=pltpu.VMEM),
            pl.BlockSpec(memory_space=pltpu.MemorySpace.HBM),
            pl.BlockSpec(memory_space=pltpu.MemorySpace.HBM),
            pl.BlockSpec(memory_space=pltpu.VMEM),
        ],
        out_specs=pl.BlockSpec(memory_space=pltpu.VMEM),
        scratch_shapes=[
            pltpu.VMEM((N_DEV, D_MODEL, H_CHUNK), BF),
            pltpu.VMEM((N_DEV, H_CHUNK, D_MODEL), BF),
            pltpu.VMEM((B_LOC, SQ, D_MODEL), BF),
            pltpu.VMEM((SQ, H_CHUNK), BF),
            pltpu.VMEM((B_LOC, HQ, DH, SKV), jnp.float32),
            pltpu.VMEM((B_LOC, HQ, DH, SKV), jnp.float32),
            pltpu.VMEM((B_LOC, HQ, DH, SKV), BF),
            pltpu.VMEM((B_LOC, HQ, DH, SKV), BF),
            pltpu.SemaphoreType.DMA,
            pltpu.SemaphoreType.DMA((2, 2)),
            pltpu.SemaphoreType.DMA((2, 2)),
            pltpu.SemaphoreType.DMA((2, 2)),
            pltpu.SemaphoreType.DMA((2, 2)),
        ],
        compiler_params=pltpu.CompilerParams(collective_id=0),
    )(x, Wq, Kt, Vt, Wo)


# device time: 28727 ns/iter; 1.0048x vs baseline; 1.0048x over previous
import jax
import jax.numpy as jnp
from jax import lax
from jax.experimental import pallas as pl
from jax.experimental.pallas import tpu as pltpu

N_DEV = 4
B_LOC = 2
SQ = 256
SKV = 256
HQ = 16
H_LOC = HQ // N_DEV
DH = 64
D_MODEL = 512
D_QK = HQ * DH
H_CHUNK = D_QK // N_DEV
BF = jnp.bfloat16


def kernel(x, Wq, K_ext, V_ext, Wo):
    pos = lax.axis_index("i")
    Kt = K_ext.transpose(0, 2, 3, 1)
    Vt = V_ext.transpose(0, 2, 3, 1)

    def body(x_ref, wq_ref, k_ref, v_ref, wo_ref, out_ref,
             wq_g, wo_g, x_scr, ctx_scr, k_scr, v_scr, kb_scr, vb_scr,
             kv_sem, wq_ss, wq_rs, wo_ss, wo_rs):
        my = lax.axis_index("i")
        left = lax.rem(my + (N_DEV - 1), N_DEV)
        right = lax.rem(my + 1, N_DEV)
        opp = lax.rem(my + 2, N_DEV)

        kv_dmas = []
        for b in range(B_LOC):
            bg = my * B_LOC + b
            kv_dmas.append(pltpu.make_async_copy(
                k_ref.at[bg], k_scr.at[b], kv_sem))
            kv_dmas.append(pltpu.make_async_copy(
                v_ref.at[bg], v_scr.at[b], kv_sem))
        for dma in kv_dmas:
            dma.start()

        barrier = pltpu.get_barrier_semaphore()
        for nbr in (left, right):
            pl.semaphore_signal(
                barrier, inc=1,
                device_id=(nbr,), device_id_type=pl.DeviceIdType.MESH,
            )
        pl.semaphore_wait(barrier, 2)

        row = lax.broadcasted_iota(jnp.int32, (SQ, SKV), 0) // 64
        col = lax.broadcasted_iota(jnp.int32, (SQ, SKV), 1) // 64
        mask = (row == col) | (col == 0) | (lax.rem(row + col, 3) == 0)
        neg = jnp.float32(-1e9)

        def mm(a, b):
            return lax.dot_general(
                a, b, dimension_numbers=(((1,), (0,)), ((), ())),
                preferred_element_type=jnp.float32,
            )

        def mm_t(a, b):
            return lax.dot_general(
                a, b, dimension_numbers=(((1,), (1,)), ((), ())),
                preferred_element_type=jnp.float32,
            )

        def do_chunk(r, wq_c, wo_c, first):
            for b in range(B_LOC):
                qp = mm(x_scr[b], wq_c)
                for hh in range(H_LOC):
                    h = lax.rem(my * H_LOC + r * H_LOC + hh, HQ)
                    q = qp[:, hh * DH:(hh + 1) * DH] * jnp.float32(0.125)
                    kt = k_scr[b, h]
                    s = mm(q, kt)
                    w = jnp.exp(jnp.where(mask, s, neg))
                    inv = 1.0 / jnp.sum(w, axis=1, keepdims=True)
                    w = w * inv
                    ctx_scr[:, hh * DH:(hh + 1) * DH] = mm_t(
                        w, v_scr[b, h]).astype(BF)
                partial = mm(ctx_scr[...], wo_c)
                if first:
                    out_ref[b, :, :] = partial
                else:
                    out_ref[b, :, :] = out_ref[b, :, :] + partial

        wq_g[0] = wq_ref[...].astype(BF)
        wo_g[0] = wo_ref[...].astype(BF)

        hop0 = []
        for d, tgt, slot in ((0, right, 3), (1, left, 1)):
            rq = pltpu.make_async_remote_copy(
                src_ref=wq_g.at[0], dst_ref=wq_g.at[slot],
                send_sem=wq_ss.at[0, d], recv_sem=wq_rs.at[0, d],
                device_id=(tgt,), device_id_type=pl.DeviceIdType.MESH,
            )
            ro = pltpu.make_async_remote_copy(
                src_ref=wo_g.at[0], dst_ref=wo_g.at[slot],
                send_sem=wo_ss.at[0, d], recv_sem=wo_rs.at[0, d],
                device_id=(tgt,), device_id_type=pl.DeviceIdType.MESH,
            )
            rq.start()
            ro.start()
            hop0 += [rq, ro]

        for b in range(B_LOC):
            x_scr[b] = x_ref[b].astype(BF)

        for dma in kv_dmas:
            dma.wait()

        do_chunk(0, wq_g[0], wo_g[0], first=True)

        for r in hop0:
            r.wait()

        hop1 = []
        rq = pltpu.make_async_remote_copy(
            src_ref=wq_g.at[3, pl.ds(0, D_MODEL // 2)],
            dst_ref=wq_g.at[2, pl.ds(0, D_MODEL // 2)],
            send_sem=wq_ss.at[1, 0], recv_sem=wq_rs.at[1, 0],
            device_id=(right,), device_id_type=pl.DeviceIdType.MESH,
        )
        ro = pltpu.make_async_remote_copy(
            src_ref=wo_g.at[3, pl.ds(0, H_CHUNK // 2)],
            dst_ref=wo_g.at[2, pl.ds(0, H_CHUNK // 2)],
            send_sem=wo_ss.at[1, 0], recv_sem=wo_rs.at[1, 0],
            device_id=(right,), device_id_type=pl.DeviceIdType.MESH,
        )
        rq.start()
        ro.start()
        hop1 += [rq, ro]
        rq = pltpu.make_async_remote_copy(
            src_ref=wq_g.at[1, pl.ds(D_MODEL // 2, D_MODEL // 2)],
            dst_ref=wq_g.at[2, pl.ds(D_MODEL // 2, D_MODEL // 2)],
            send_sem=wq_ss.at[1, 1], recv_sem=wq_rs.at[1, 1],
            device_id=(left,), device_id_type=pl.DeviceIdType.MESH,
        )
        ro = pltpu.make_async_remote_copy(
            src_ref=wo_g.at[1, pl.ds(H_CHUNK // 2, H_CHUNK // 2)],
            dst_ref=wo_g.at[2, pl.ds(H_CHUNK // 2, H_CHUNK // 2)],
            send_sem=wo_ss.at[1, 1], recv_sem=wo_rs.at[1, 1],
            device_id=(left,), device_id_type=pl.DeviceIdType.MESH,
        )
        rq.start()
        ro.start()
        hop1 += [rq, ro]

        do_chunk(3, wq_g[3], wo_g[3], first=False)
        do_chunk(1, wq_g[1], wo_g[1], first=False)

        for r in hop1:
            r.wait()

        do_chunk(2, wq_g[2], wo_g[2], first=False)

    return pl.pallas_call(
        body,
        out_shape=jax.ShapeDtypeStruct((B_LOC, SQ, D_MODEL), jnp.float32),
        in_specs=[
            pl.BlockSpec(memory_space=pltpu.VMEM),
            pl.BlockSpec(memory_space=pltpu.VMEM),
            pl.BlockSpec(memory_space=pltpu.MemorySpace.HBM),
            pl.BlockSpec(memory_space=pltpu.MemorySpace.HBM),
            pl.BlockSpec(memory_space=pltpu.VMEM),
        ],
        out_specs=pl.BlockSpec(memory_space=pltpu.VMEM),
        scratch_shapes=[
            pltpu.VMEM((N_DEV, D_MODEL, H_CHUNK), BF),
            pltpu.VMEM((N_DEV, H_CHUNK, D_MODEL), BF),
            pltpu.VMEM((B_LOC, SQ, D_MODEL), BF),
            pltpu.VMEM((SQ, H_CHUNK), BF),
            pltpu.VMEM((B_LOC, HQ, DH, SKV), jnp.float32),
            pltpu.VMEM((B_LOC, HQ, DH, SKV), jnp.float32),
            pltpu.VMEM((B_LOC, HQ, DH, SKV), BF),
            pltpu.VMEM((B_LOC, HQ, DH, SKV), BF),
            pltpu.SemaphoreType.DMA,
            pltpu.SemaphoreType.DMA((2, 2)),
            pltpu.SemaphoreType.DMA((2, 2)),
            pltpu.SemaphoreType.DMA((2, 2)),
            pltpu.SemaphoreType.DMA((2, 2)),
        ],
        compiler_params=pltpu.CompilerParams(collective_id=0),
    )(x, Wq, Kt, Vt, Wo)


# device time: 22886 ns/iter; 1.2613x vs baseline; 1.2552x over previous
import jax
import jax.numpy as jnp
from jax import lax
from jax.experimental import pallas as pl
from jax.experimental.pallas import tpu as pltpu

N_DEV = 4
B_LOC = 2
SQ = 256
SKV = 256
HQ = 16
H_LOC = HQ // N_DEV
DH = 64
D_MODEL = 512
D_QK = HQ * DH
H_CHUNK = D_QK // N_DEV
BF = jnp.bfloat16


def kernel(x, Wq, K_ext, V_ext, Wo):
    pos = lax.axis_index("i")
    K_loc = lax.dynamic_slice_in_dim(K_ext, pos * B_LOC, B_LOC, axis=0)
    V_loc = lax.dynamic_slice_in_dim(V_ext, pos * B_LOC, B_LOC, axis=0)

    def arrange(a):
        a = a.reshape(B_LOC, SKV, N_DEV, H_LOC, DH)
        return a.transpose(2, 0, 3, 1, 4).astype(BF)

    Kt = arrange(K_loc)
    Vt = arrange(V_loc)

    def body(x_ref, wq_ref, k_ref, v_ref, wo_ref, out_ref,
             wq_g, wo_g, x_scr, ctx_scr, wq_ss, wq_rs, wo_ss, wo_rs):
        my = lax.axis_index("i")
        left = lax.rem(my + (N_DEV - 1), N_DEV)
        right = lax.rem(my + 1, N_DEV)
        opp = lax.rem(my + 2, N_DEV)

        barrier = pltpu.get_barrier_semaphore()
        for nbr in (left, right):
            pl.semaphore_signal(
                barrier, inc=1,
                device_id=(nbr,), device_id_type=pl.DeviceIdType.MESH,
            )
        pl.semaphore_wait(barrier, 2)

        row = lax.broadcasted_iota(jnp.int32, (SQ, SKV), 0) // 64
        col = lax.broadcasted_iota(jnp.int32, (SQ, SKV), 1) // 64
        mask = (row == col) | (col == 0) | (lax.rem(row + col, 3) == 0)
        neg = jnp.float32(-1e9)

        def mm(a, b):
            return lax.dot_general(
                a, b, dimension_numbers=(((1,), (0,)), ((), ())),
                preferred_element_type=jnp.float32,
            )

        def mm_t(a, b):
            return lax.dot_general(
                a, b, dimension_numbers=(((1,), (1,)), ((), ())),
                preferred_element_type=jnp.float32,
            )

        def do_chunk(p, wq_c, wo_c, first):
            for b in range(B_LOC):
                qp = mm(x_scr[b], wq_c)
                for hh in range(H_LOC):
                    q = (qp[:, hh * DH:(hh + 1) * DH]
                         * jnp.float32(0.125)).astype(BF)
                    k = k_ref[p, b, hh]
                    s = mm_t(q, k)
                    w = jnp.exp(jnp.where(mask, s, neg))
                    inv = 1.0 / jnp.sum(w, axis=1, keepdims=True)
                    w = (w * inv).astype(BF)
                    ctx_scr[:, hh * DH:(hh + 1) * DH] = mm(
                        w, v_ref[p, b, hh]).astype(BF)
                partial = mm(ctx_scr[...], wo_c)
                if first:
                    out_ref[b, :, :] = partial
                else:
                    out_ref[b, :, :] = out_ref[b, :, :] + partial

        wq_g[my] = wq_ref[...].astype(BF)
        wo_g[my] = wo_ref[...].astype(BF)

        hop0 = []
        for d, tgt in ((0, right), (1, left)):
            rq = pltpu.make_async_remote_copy(
                src_ref=wq_g.at[my], dst_ref=wq_g.at[my],
                send_sem=wq_ss.at[0, d], recv_sem=wq_rs.at[0, d],
                device_id=(tgt,), device_id_type=pl.DeviceIdType.MESH,
            )
            ro = pltpu.make_async_remote_copy(
                src_ref=wo_g.at[my], dst_ref=wo_g.at[my],
                send_sem=wo_ss.at[0, d], recv_sem=wo_rs.at[0, d],
                device_id=(tgt,), device_id_type=pl.DeviceIdType.MESH,
            )
            rq.start()
            ro.start()
            hop0 += [rq, ro]

        for b in range(B_LOC):
            x_scr[b] = x_ref[b].astype(BF)

        do_chunk(my, wq_g[my], wo_g[my], first=True)

        for r in hop0:
            r.wait()

        hop1 = []
        rq = pltpu.make_async_remote_copy(
            src_ref=wq_g.at[left, pl.ds(0, D_MODEL // 2)],
            dst_ref=wq_g.at[left, pl.ds(0, D_MODEL // 2)],
            send_sem=wq_ss.at[1, 0], recv_sem=wq_rs.at[1, 0],
            device_id=(right,), device_id_type=pl.DeviceIdType.MESH,
        )
        ro = pltpu.make_async_remote_copy(
            src_ref=wo_g.at[left, pl.ds(0, H_CHUNK // 2)],
            dst_ref=wo_g.at[left, pl.ds(0, H_CHUNK // 2)],
            send_sem=wo_ss.at[1, 0], recv_sem=wo_rs.at[1, 0],
            device_id=(right,), device_id_type=pl.DeviceIdType.MESH,
        )
        rq.start()
        ro.start()
        hop1 += [rq, ro]
        rq = pltpu.make_async_remote_copy(
            src_ref=wq_g.at[right, pl.ds(D_MODEL // 2, D_MODEL // 2)],
            dst_ref=wq_g.at[right, pl.ds(D_MODEL // 2, D_MODEL // 2)],
            send_sem=wq_ss.at[1, 1], recv_sem=wq_rs.at[1, 1],
            device_id=(left,), device_id_type=pl.DeviceIdType.MESH,
        )
        ro = pltpu.make_async_remote_copy(
            src_ref=wo_g.at[right, pl.ds(H_CHUNK // 2, H_CHUNK // 2)],
            dst_ref=wo_g.at[right, pl.ds(H_CHUNK // 2, H_CHUNK // 2)],
            send_sem=wo_ss.at[1, 1], recv_sem=wo_rs.at[1, 1],
            device_id=(left,), device_id_type=pl.DeviceIdType.MESH,
        )
        rq.start()
        ro.start()
        hop1 += [rq, ro]

        do_chunk(left, wq_g[left], wo_g[left], first=False)
        do_chunk(right, wq_g[right], wo_g[right], first=False)

        for r in hop1:
            r.wait()

        do_chunk(opp, wq_g[opp], wo_g[opp], first=False)

    return pl.pallas_call(
        body,
        out_shape=jax.ShapeDtypeStruct((B_LOC, SQ, D_MODEL), jnp.float32),
        in_specs=[pl.BlockSpec(memory_space=pltpu.VMEM)] * 5,
        out_specs=pl.BlockSpec(memory_space=pltpu.VMEM),
        scratch_shapes=[
            pltpu.VMEM((N_DEV, D_MODEL, H_CHUNK), BF),
            pltpu.VMEM((N_DEV, H_CHUNK, D_MODEL), BF),
            pltpu.VMEM((B_LOC, SQ, D_MODEL), BF),
            pltpu.VMEM((SQ, H_CHUNK), BF),
            pltpu.SemaphoreType.DMA((2, 2)),
            pltpu.SemaphoreType.DMA((2, 2)),
            pltpu.SemaphoreType.DMA((2, 2)),
            pltpu.SemaphoreType.DMA((2, 2)),
        ],
        compiler_params=pltpu.CompilerParams(collective_id=0),
    )(x, Wq, Kt, Vt, Wo)


# device time: 22808 ns/iter; 1.2656x vs baseline; 1.0034x over previous
import jax
import jax.numpy as jnp
from jax import lax
from jax.experimental import pallas as pl
from jax.experimental.pallas import tpu as pltpu

N_DEV = 4
B_LOC = 2
SQ = 256
SKV = 256
HQ = 16
H_LOC = HQ // N_DEV
DH = 64
D_MODEL = 512
D_QK = HQ * DH
H_CHUNK = D_QK // N_DEV
BF = jnp.bfloat16


def kernel(x, Wq, K_ext, V_ext, Wo):
    pos = lax.axis_index("i")
    K_loc = lax.dynamic_slice_in_dim(K_ext, pos * B_LOC, B_LOC, axis=0)
    V_loc = lax.dynamic_slice_in_dim(V_ext, pos * B_LOC, B_LOC, axis=0)

    def arrange(a):
        a = a.reshape(B_LOC, SKV, N_DEV, H_LOC, DH)
        return a.transpose(2, 0, 3, 1, 4).astype(BF)

    Kt = arrange(K_loc)
    Vt = arrange(V_loc)

    def body(x_ref, wq_ref, k_ref, v_ref, wo_ref, out_ref,
             wq_g, wo_g, x_scr, ctx_scr, wq_ss, wq_rs, wo_ss, wo_rs):
        my = lax.axis_index("i")
        left = lax.rem(my + (N_DEV - 1), N_DEV)
        right = lax.rem(my + 1, N_DEV)
        opp = lax.rem(my + 2, N_DEV)

        barrier = pltpu.get_barrier_semaphore()
        for nbr in (left, right):
            pl.semaphore_signal(
                barrier, inc=1,
                device_id=(nbr,), device_id_type=pl.DeviceIdType.MESH,
            )
        pl.semaphore_wait(barrier, 2)

        row = lax.broadcasted_iota(jnp.int32, (SQ, SKV), 0) // 64
        col = lax.broadcasted_iota(jnp.int32, (SQ, SKV), 1) // 64
        mask = (row == col) | (col == 0) | (lax.rem(row + col, 3) == 0)
        neg = jnp.float32(-1e9)

        def mm(a, b):
            return lax.dot_general(
                a, b, dimension_numbers=(((1,), (0,)), ((), ())),
                preferred_element_type=jnp.float32,
            )

        def mm_t(a, b):
            return lax.dot_general(
                a, b, dimension_numbers=(((1,), (1,)), ((), ())),
                preferred_element_type=jnp.float32,
            )

        def do_chunk(p, wq_c, wo_c, first):
            for b in range(B_LOC):
                qp = mm(x_scr[b], wq_c)
                for hh in range(H_LOC):
                    q = (qp[:, hh * DH:(hh + 1) * DH]
                         * jnp.float32(0.125)).astype(BF)
                    k = k_ref[p, b, hh]
                    s = mm_t(q, k)
                    w = jnp.exp(jnp.where(mask, s, neg))
                    inv = 1.0 / jnp.sum(w, axis=1, keepdims=True)
                    w = (w * inv).astype(BF)
                    ctx_scr[:, hh * DH:(hh + 1) * DH] = mm(
                        w, v_ref[p, b, hh]).astype(BF)
                partial = mm(ctx_scr[...], wo_c)
                if first:
                    out_ref[b, :, :] = partial
                else:
                    out_ref[b, :, :] = out_ref[b, :, :] + partial

        wq_g[my] = wq_ref[...].astype(BF)
        wo_g[my] = wo_ref[...].astype(BF)

        hop0_wq, hop0_wo = [], []
        for d, tgt in ((0, right), (1, left)):
            rq = pltpu.make_async_remote_copy(
                src_ref=wq_g.at[my], dst_ref=wq_g.at[my],
                send_sem=wq_ss.at[0, d], recv_sem=wq_rs.at[0, d],
                device_id=(tgt,), device_id_type=pl.DeviceIdType.MESH,
            )
            rq.start()
            hop0_wq.append(rq)
        for d, tgt in ((0, right), (1, left)):
            ro = pltpu.make_async_remote_copy(
                src_ref=wo_g.at[my], dst_ref=wo_g.at[my],
                send_sem=wo_ss.at[0, d], recv_sem=wo_rs.at[0, d],
                device_id=(tgt,), device_id_type=pl.DeviceIdType.MESH,
            )
            ro.start()
            hop0_wo.append(ro)

        for b in range(B_LOC):
            x_scr[b] = x_ref[b].astype(BF)

        do_chunk(my, wq_g[my], wo_g[my], first=True)

        for r in hop0_wq:
            r.wait()
        hop1 = []
        rq = pltpu.make_async_remote_copy(
            src_ref=wq_g.at[left, pl.ds(0, D_MODEL // 2)],
            dst_ref=wq_g.at[left, pl.ds(0, D_MODEL // 2)],
            send_sem=wq_ss.at[1, 0], recv_sem=wq_rs.at[1, 0],
            device_id=(right,), device_id_type=pl.DeviceIdType.MESH,
        )
        rq.start()
        hop1.append(rq)
        rq = pltpu.make_async_remote_copy(
            src_ref=wq_g.at[right, pl.ds(D_MODEL // 2, D_MODEL // 2)],
            dst_ref=wq_g.at[right, pl.ds(D_MODEL // 2, D_MODEL // 2)],
            send_sem=wq_ss.at[1, 1], recv_sem=wq_rs.at[1, 1],
            device_id=(left,), device_id_type=pl.DeviceIdType.MESH,
        )
        rq.start()
        hop1.append(rq)

        for r in hop0_wo:
            r.wait()
        ro = pltpu.make_async_remote_copy(
            src_ref=wo_g.at[left, pl.ds(0, H_CHUNK // 2)],
            dst_ref=wo_g.at[left, pl.ds(0, H_CHUNK // 2)],
            send_sem=wo_ss.at[1, 0], recv_sem=wo_rs.at[1, 0],
            device_id=(right,), device_id_type=pl.DeviceIdType.MESH,
        )
        ro.start()
        hop1.append(ro)
        ro = pltpu.make_async_remote_copy(
            src_ref=wo_g.at[right, pl.ds(H_CHUNK // 2, H_CHUNK // 2)],
            dst_ref=wo_g.at[right, pl.ds(H_CHUNK // 2, H_CHUNK // 2)],
            send_sem=wo_ss.at[1, 1], recv_sem=wo_rs.at[1, 1],
            device_id=(left,), device_id_type=pl.DeviceIdType.MESH,
        )
        ro.start()
        hop1.append(ro)

        do_chunk(left, wq_g[left], wo_g[left], first=False)
        do_chunk(right, wq_g[right], wo_g[right], first=False)

        for r in hop1:
            r.wait()

        do_chunk(opp, wq_g[opp], wo_g[opp], first=False)

    return pl.pallas_call(
        body,
        out_shape=jax.ShapeDtypeStruct((B_LOC, SQ, D_MODEL), jnp.float32),
        in_specs=[pl.BlockSpec(memory_space=pltpu.VMEM)] * 5,
        out_specs=pl.BlockSpec(memory_space=pltpu.VMEM),
        scratch_shapes=[
            pltpu.VMEM((N_DEV, D_MODEL, H_CHUNK), BF),
            pltpu.VMEM((N_DEV, H_CHUNK, D_MODEL), BF),
            pltpu.VMEM((B_LOC, SQ, D_MODEL), BF),
            pltpu.VMEM((SQ, H_CHUNK), BF),
            pltpu.SemaphoreType.DMA((2, 2)),
            pltpu.SemaphoreType.DMA((2, 2)),
            pltpu.SemaphoreType.DMA((2, 2)),
            pltpu.SemaphoreType.DMA((2, 2)),
        ],
        compiler_params=pltpu.CompilerParams(collective_id=0),
    )(x, Wq, Kt, Vt, Wo)
